# baseline (speedup 1.0000x reference)
import jax
import jax.numpy as jnp
from jax import lax
from jax.experimental import pallas as pl
from jax.experimental.pallas import tpu as pltpu
from jax.experimental.pallas import tpu_sc as plsc

_B = 4096
_S = 200
_K = 4
_NC = 2
_NS = 16
_NW = _NC * _NS
_ROWS_W = _B // _NW
_HALVES = 2
_ROWS_H = _ROWS_W // _HALVES
_ELEMS_H = _ROWS_H * _S
_PAIRS_H = _ROWS_H // 2
_CHUNKS = (2 * _S) // 16
_L = 16
_IGNORED = -1


def _body(x_hbm, xe_hbm, w0_hbm, w1_hbm, w2_hbm, w3_hbm, out_hbm,
          x_v, xe_v, tab_v, i0_v, i1_v, i2_v, i3_v, out_v, sem):
  wid = lax.axis_index("s") * _NC + lax.axis_index("c")

  pltpu.sync_copy(xe_hbm.at[pl.ds(wid * (_ROWS_W * 3), _ROWS_W * 3)], xe_v)

  iota = lax.iota(jnp.int32, _L)

  def tab_step(r, carry):
    words = (r * _L + iota) * 3 + 2
    strat = plsc.load_gather(xe_v, [words])
    tab_v[pl.ds(r * _L, _L)] = jnp.minimum(strat, _K - 1)
    return carry

  lax.fori_loop(0, _ROWS_W // _L, tab_step, 0)

  for h in range(_HALVES):
    row0 = wid * _ROWS_W + h * _ROWS_H
    pltpu.sync_copy(x_hbm.at[pl.ds(row0 * _S * 2, _ELEMS_H * 2)], x_v)

    def pair_step(j, carry):
      off = (2 * _S) * j
      for c in range(_CHUNKS):
        p = iota + (c * _L)
        row = 2 * j + (p >= _S).astype(jnp.int32)
        xi = plsc.load_gather(x_v, [2 * p + 2 * off])
        tb = plsc.load_gather(tab_v, [row + (h * _ROWS_H)])
        sl = pl.ds(off + c * _L, _L)
        i0_v[sl] = jnp.where(tb == 0, xi, _IGNORED)
        i1_v[sl] = jnp.where(tb == 1, xi, _IGNORED)
        i2_v[sl] = jnp.where(tb == 2, xi, _IGNORED)
        i3_v[sl] = jnp.where(tb == 3, xi, _IGNORED)
      return carry

    lax.fori_loop(0, _PAIRS_H, pair_step, 0)

    cps = [
        pltpu.async_copy(
            w.at[plsc.Indices(iv, ignored_value=_IGNORED)], out_v, sem)
        for w, iv in ((w0_hbm, i0_v), (w1_hbm, i1_v),
                      (w2_hbm, i2_v), (w3_hbm, i3_v))
    ]
    for cp in cps:
      cp.wait()

    pltpu.sync_copy(
        out_v, out_hbm.at[pl.ds(wid * (_ROWS_W * _S) + h * _ELEMS_H,
                                _ELEMS_H)])


@jax.jit
def _stratified_gather(x_flat, xe_flat, w0, w1, w2, w3):
  mesh = plsc.VectorSubcoreMesh(
      core_axis_name="c", subcore_axis_name="s", num_cores=_NC,
      num_subcores=_NS)
  run = pl.kernel(
      _body,
      out_type=jax.ShapeDtypeStruct((_B * _S,), jnp.float32),
      mesh=mesh,
      compiler_params=pltpu.CompilerParams(needs_layout_passes=False),
      scratch_types=[
          pltpu.VMEM((_ELEMS_H * 2,), jnp.int32),
          pltpu.VMEM((_ROWS_W * 3,), jnp.int32),
          pltpu.VMEM((_ROWS_W,), jnp.int32),
          pltpu.VMEM((_ELEMS_H,), jnp.int32),
          pltpu.VMEM((_ELEMS_H,), jnp.int32),
          pltpu.VMEM((_ELEMS_H,), jnp.int32),
          pltpu.VMEM((_ELEMS_H,), jnp.int32),
          pltpu.VMEM((_ELEMS_H,), jnp.float32),
          pltpu.SemaphoreType.DMA,
      ],
  )
  return run(x_flat, xe_flat, w0, w1, w2, w3)


def kernel(x, x_extra, W0, W1, W2, W3):
  out = _stratified_gather(
      x.reshape(-1), x_extra.reshape(-1), W0.reshape(-1), W1.reshape(-1),
      W2.reshape(-1), W3.reshape(-1))
  return out.reshape(_B, _S, 1)


# speedup vs baseline: 5.1708x; 5.1708x over previous
import jax
import jax.numpy as jnp
from jax import lax
from jax.experimental import pallas as pl
from jax.experimental.pallas import tpu as pltpu
from jax.experimental.pallas import tpu_sc as plsc

_B = 4096
_S = 200
_K = 4
_NC = 2
_NS = 16
_NW = _NC * _NS
_ROWS_W = _B // _NW
_HALVES = 2
_ROWS_H = _ROWS_W // _HALVES
_ELEMS_H = _ROWS_H * _S
_PAIRS_H = _ROWS_H // 2
_CHUNKS = (2 * _S) // 16
_L = 16
_IGNORED = -1


def _build_body(idx_hbm, xe_hbm, i0_hbm, i1_hbm, i2_hbm, i3_hbm,
                x_v, xe_v, tab_v, i0_v, i1_v, i2_v, i3_v, sem):
  wid = lax.axis_index("s") * _NC + lax.axis_index("c")

  pltpu.sync_copy(xe_hbm.at[pl.ds(wid * (_ROWS_W * 3), _ROWS_W * 3)], xe_v)

  iota = lax.iota(jnp.int32, _L)

  def tab_step(r, carry):
    words = (r * _L + iota) * 3 + 2
    strat = plsc.load_gather(xe_v, [words])
    tab_v[pl.ds(r * _L, _L)] = jnp.minimum(strat, _K - 1)
    return carry

  lax.fori_loop(0, _ROWS_W // _L, tab_step, 0)

  cols = [c * _L for c in range(_S // _L)] + [_S - _L]

  for h in range(_HALVES):
    row0 = wid * _ROWS_W + h * _ROWS_H
    pltpu.sync_copy(idx_hbm.at[pl.ds(row0, _ROWS_H), :], x_v)

    def row_step(r, carry, h=h):
      tb = plsc.load_gather(tab_v, [jnp.full((_L,), h * _ROWS_H, jnp.int32)
                                    + r])
      m0, m1, m2, m3 = tb == 0, tb == 1, tb == 2, tb == 3
      for col in cols:
        xi = x_v[r, pl.ds(col, _L)]
        sl = pl.ds(r * _S + col, _L)
        i0_v[sl] = jnp.where(m0, xi, _IGNORED)
        i1_v[sl] = jnp.where(m1, xi, _IGNORED)
        i2_v[sl] = jnp.where(m2, xi, _IGNORED)
        i3_v[sl] = jnp.where(m3, xi, _IGNORED)
      return carry

    lax.fori_loop(0, _ROWS_H, row_step, 0)

    dst = pl.ds(wid * (_ROWS_W * _S) + h * _ELEMS_H, _ELEMS_H)
    cps = [
        pltpu.async_copy(iv, ih.at[dst], sem)
        for iv, ih in ((i0_v, i0_hbm), (i1_v, i1_hbm),
                       (i2_v, i2_hbm), (i3_v, i3_hbm))
    ]
    for cp in cps:
      cp.wait()


def _gather_first_body(i_hbm, w_hbm, out_hbm, idx0_v, out0_v, idx1_v,
                       out1_v, sem0, sem1):
  wid = lax.axis_index("s") * _NC + lax.axis_index("c")
  srcs = [pl.ds(wid * (_ROWS_W * _S) + h * _ELEMS_H, _ELEMS_H)
          for h in range(_HALVES)]
  bufs = [(idx0_v, out0_v), (idx1_v, out1_v)]
  sems = [sem0, sem1]
  loads = [pltpu.async_copy(i_hbm.at[srcs[h]], bufs[h][0], sems[h])
           for h in range(_HALVES)]
  gathers = []
  for h in range(_HALVES):
    loads[h].wait()
    gathers.append(pltpu.async_copy(
        w_hbm.at[plsc.Indices(bufs[h][0], ignored_value=_IGNORED)],
        bufs[h][1], sems[h]))
  for h in range(_HALVES):
    gathers[h].wait()
    pltpu.sync_copy(bufs[h][1], out_hbm.at[srcs[h]])


def _gather_next_body(prev_hbm, i_hbm, w_hbm, out_hbm, idx0_v, out0_v,
                      idx1_v, out1_v, sem0, sem1):
  wid = lax.axis_index("s") * _NC + lax.axis_index("c")
  srcs = [pl.ds(wid * (_ROWS_W * _S) + h * _ELEMS_H, _ELEMS_H)
          for h in range(_HALVES)]
  bufs = [(idx0_v, out0_v), (idx1_v, out1_v)]
  sems = [sem0, sem1]
  loads = []
  for h in range(_HALVES):
    loads.append((
        pltpu.async_copy(prev_hbm.at[srcs[h]], bufs[h][1], sems[h]),
        pltpu.async_copy(i_hbm.at[srcs[h]], bufs[h][0], sems[h])))
  gathers = []
  for h in range(_HALVES):
    loads[h][0].wait()
    loads[h][1].wait()
    gathers.append(pltpu.async_copy(
        w_hbm.at[plsc.Indices(bufs[h][0], ignored_value=_IGNORED)],
        bufs[h][1], sems[h]))
  for h in range(_HALVES):
    gathers[h].wait()
    pltpu.sync_copy(bufs[h][1], out_hbm.at[srcs[h]])


@jax.jit
def _stratified_gather(idx_flat, xe_flat, w0, w1, w2, w3):
  mesh = plsc.VectorSubcoreMesh(
      core_axis_name="c", subcore_axis_name="s", num_cores=_NC,
      num_subcores=_NS)
  cp = pltpu.CompilerParams(needs_layout_passes=False)
  ibuf = jax.ShapeDtypeStruct((_B * _S,), jnp.int32)
  obuf = jax.ShapeDtypeStruct((_B * _S,), jnp.float32)
  build = pl.kernel(
      _build_body,
      out_type=(ibuf, ibuf, ibuf, ibuf),
      mesh=mesh,
      compiler_params=cp,
      scratch_types=[
          pltpu.VMEM((_ROWS_H, _S), jnp.int32),
          pltpu.VMEM((_ROWS_W * 3,), jnp.int32),
          pltpu.VMEM((_ROWS_W,), jnp.int32),
          pltpu.VMEM((_ELEMS_H,), jnp.int32),
          pltpu.VMEM((_ELEMS_H,), jnp.int32),
          pltpu.VMEM((_ELEMS_H,), jnp.int32),
          pltpu.VMEM((_ELEMS_H,), jnp.int32),
          pltpu.SemaphoreType.DMA,
      ],
  )
  i0, i1, i2, i3 = build(idx_flat, xe_flat)
  gscratch = [
      pltpu.VMEM((_ELEMS_H,), jnp.int32),
      pltpu.VMEM((_ELEMS_H,), jnp.float32),
      pltpu.VMEM((_ELEMS_H,), jnp.int32),
      pltpu.VMEM((_ELEMS_H,), jnp.float32),
      pltpu.SemaphoreType.DMA,
      pltpu.SemaphoreType.DMA,
  ]
  gfirst = pl.kernel(_gather_first_body, out_type=obuf, mesh=mesh,
                     compiler_params=cp, scratch_types=gscratch)
  gnext = pl.kernel(_gather_next_body, out_type=obuf, mesh=mesh,
                    compiler_params=cp, scratch_types=gscratch)
  out = gfirst(i0, w0)
  out = gnext(out, i1, w1)
  out = gnext(out, i2, w2)
  out = gnext(out, i3, w3)
  return out


def kernel(x, x_extra, W0, W1, W2, W3):
  idx = x[:, :, 0]
  out = _stratified_gather(
      idx, x_extra.reshape(-1), W0.reshape(-1), W1.reshape(-1),
      W2.reshape(-1), W3.reshape(-1))
  return out.reshape(_B, _S, 1)
